# baseline (speedup 1.0000x reference)
import jax
import jax.numpy as jnp
from jax import lax
from jax.experimental import pallas as pl
from jax.experimental.pallas import tpu as pltpu
from jax.experimental.pallas import tpu_sc as plsc

TOTAL_BUCKETS = 16 * (2 ** 16)
EMB_DIM = 32
B = 16384
NUM_NEG = 8
M = 16

NC = 2
NS = 16
NW = NC * NS
L = 16

EL = B // NW
CH = 64
NCHUNK = EL // CH
BT = B // 128
CT_PER_W = EL // 128


def _stripe_set(table, idxT, band0, out_sliced, idx_v, rows_v, stage_v,
                sems, wid):
    pltpu.sync_copy(
        idxT.at[pl.ds(band0, 2), pl.ds(wid * CT_PER_W, CT_PER_W)], idx_v)

    def issue(c, buf):
        ct = c // 2
        half = c % 2
        for h in range(M):
            pltpu.async_copy(table.at[idx_v.at[h // 8, ct, h % 8, half]],
                             rows_v.at[buf, h], sems[buf])

    def wait_pool(c, buf):
        ct = c // 2
        half = c % 2
        for h in range(M):
            pltpu.make_async_copy(table.at[idx_v.at[h // 8, ct, h % 8, half]],
                                  rows_v.at[buf, h], sems[buf]).wait()

        def pool_g(g, _):
            for el in range(8):
                e = g * 8 + el
                acc0 = rows_v[buf, 0, e, 0:L]
                acc1 = rows_v[buf, 0, e, L:EMB_DIM]
                for h in range(1, M):
                    acc0 = acc0 + rows_v[buf, h, e, 0:L]
                    acc1 = acc1 + rows_v[buf, h, e, L:EMB_DIM]
                row = c * CH + e
                stage_v[row, 0:L] = acc0 * (1.0 / M)
                stage_v[row, L:EMB_DIM] = acc1 * (1.0 / M)
            return 0

        lax.fori_loop(0, CH // 8, pool_g, 0)

    issue(0, 0)
    issue(1, 1)

    def pair(p, _):
        for b in range(2):
            c = 2 * p + b
            wait_pool(c, b)

            @pl.when(c + 2 < NCHUNK)
            def _():
                issue(c + 2, b)
        return 0

    lax.fori_loop(0, NCHUNK // 2, pair, 0)
    pltpu.sync_copy(stage_v, out_sliced)


def _sc_body(srcT, posT, negT, w_src, w_tgt,
             su_out, tp_out, tn_out,
             idx_v, rows_v, stage_v, sem0, sem1):
    wid = lax.axis_index("s") * NC + lax.axis_index("c")
    sems = (sem0, sem1)
    base = wid * EL

    _stripe_set(w_src, srcT, 0, su_out.at[pl.ds(base, EL)],
                idx_v, rows_v, stage_v, sems, wid)
    _stripe_set(w_tgt, posT, 0, tp_out.at[pl.ds(base, EL)],
                idx_v, rows_v, stage_v, sems, wid)

    def neg_body(n, _):
        _stripe_set(w_tgt, negT, n * 2, tn_out.at[n].at[pl.ds(base, EL)],
                    idx_v, rows_v, stage_v, sems, wid)
        return 0

    lax.fori_loop(0, NUM_NEG, neg_body, 0)


def _make_sc_pool():
    mesh = plsc.VectorSubcoreMesh(core_axis_name="c", subcore_axis_name="s",
                                  num_cores=NC, num_subcores=NS)
    return pl.kernel(
        _sc_body,
        out_type=[
            jax.ShapeDtypeStruct((B, EMB_DIM), jnp.float32),
            jax.ShapeDtypeStruct((B, EMB_DIM), jnp.float32),
            jax.ShapeDtypeStruct((NUM_NEG, B, EMB_DIM), jnp.float32),
        ],
        mesh=mesh,
        scratch_types=[
            pltpu.VMEM((2, CT_PER_W, 8, 2, CH), jnp.int32),
            pltpu.VMEM((2, M, CH, EMB_DIM), jnp.float32),
            pltpu.VMEM((EL, EMB_DIM), jnp.float32),
            pltpu.SemaphoreType.DMA,
            pltpu.SemaphoreType.DMA,
        ],
        compiler_params=pltpu.CompilerParams(use_tc_tiling_on_sc=False),
    )


def _softplus(x):
    return jnp.maximum(x, 0.0) + jnp.log1p(jnp.exp(-jnp.abs(x)))


def _loss_body(su_ref, tp_ref, tn_ref, out_ref):
    su = su_ref[...]
    tp = tp_ref[...]
    acc = _softplus(-jnp.sum(su * tp, axis=1))
    for n in range(NUM_NEG):
        acc = acc + _softplus(jnp.sum(su * tn_ref[n], axis=1))
    tot = jnp.sum(acc).reshape(1, 1)

    @pl.when(pl.program_id(0) == 0)
    def _():
        out_ref[...] = jnp.zeros((1, 1), jnp.float32)

    out_ref[...] += tot


_TC_BLOCK = 512


def _make_tc_loss():
    grid = (B // _TC_BLOCK,)
    return pl.pallas_call(
        _loss_body,
        grid=grid,
        in_specs=[
            pl.BlockSpec((_TC_BLOCK, EMB_DIM), lambda i: (i, 0)),
            pl.BlockSpec((_TC_BLOCK, EMB_DIM), lambda i: (i, 0)),
            pl.BlockSpec((NUM_NEG, _TC_BLOCK, EMB_DIM), lambda i: (0, i, 0)),
        ],
        out_specs=pl.BlockSpec((1, 1), lambda i: (0, 0)),
        out_shape=jax.ShapeDtypeStruct((1, 1), jnp.float32),
    )


@jax.jit
def kernel(src_hashes, pos_dst_hashes, neg_dst_hashes, W_src, W_tgt):
    def native5(x2d):
        return (x2d.astype(jnp.int32).T.reshape(2, 8, BT, 128)
                .transpose(0, 2, 1, 3).reshape(2, BT, 8, 2, CH))

    srcT = native5(src_hashes)
    posT = native5(pos_dst_hashes)
    negT = (neg_dst_hashes.astype(jnp.int32).transpose(1, 2, 0)
            .reshape(NUM_NEG, 2, 8, BT, 128).transpose(0, 1, 3, 2, 4)
            .reshape(NUM_NEG * 2, BT, 8, 2, CH))

    su, tp, tn = _make_sc_pool()(srcT, posT, negT, W_src, W_tgt)
    tot = _make_tc_loss()(su, tp, tn)
    return tot[0, 0] / B


# speedup vs baseline: 1.0190x; 1.0190x over previous
import jax
import jax.numpy as jnp
from jax import lax
from jax.experimental import pallas as pl
from jax.experimental.pallas import tpu as pltpu
from jax.experimental.pallas import tpu_sc as plsc

TOTAL_BUCKETS = 16 * (2 ** 16)
EMB_DIM = 32
B = 16384
NUM_NEG = 8
M = 16

NC = 2
NS = 16
NW = NC * NS
L = 16

EL = B // NW
CH = 64
NCHUNK = EL // CH
BT = B // 128
CT_PER_W = EL // 128


def _emit_stripe_set(table, load_idx, store_out, idx_v, rows_v, stage_v,
                     sems):
    load_idx()

    def issue(c, buf):
        ct = c // 2
        half = c % 2
        for h in range(M):
            pltpu.async_copy(table.at[idx_v.at[h // 8, ct, h % 8, half]],
                             rows_v.at[buf, h], sems[buf])

    def wait_pool(c, buf):
        ct = c // 2
        half = c % 2
        for h in range(M):
            pltpu.make_async_copy(table.at[idx_v.at[h // 8, ct, h % 8, half]],
                                  rows_v.at[buf, h], sems[buf]).wait()

        def pool_g(g, _):
            for el in range(4):
                e = g * 4 + el
                acc0 = rows_v[buf, 0, e, 0:L]
                acc1 = rows_v[buf, 0, e, L:EMB_DIM]
                for h in range(1, M):
                    acc0 = acc0 + rows_v[buf, h, e, 0:L]
                    acc1 = acc1 + rows_v[buf, h, e, L:EMB_DIM]
                row = c * CH + e
                stage_v[row, 0:L] = acc0 * (1.0 / M)
                stage_v[row, L:EMB_DIM] = acc1 * (1.0 / M)
            return 0

        lax.fori_loop(0, CH // 4, pool_g, 0)

    issue(0, 0)
    issue(1, 1)

    def pair(p, _):
        for b in range(2):
            c = 2 * p + b
            wait_pool(c, b)

            @pl.when(c + 2 < NCHUNK)
            def _():
                issue(c + 2, b)
        return 0

    lax.fori_loop(0, NCHUNK // 2, pair, 0)
    store_out()


def _sc_body(srcT, posT, ng0, ng1, ng2, ng3, ng4, ng5, ng6, ng7,
             w_src, w_tgt,
             su_out, tp_out, tn_out,
             idx_v, rows_v, stage_v, sem0, sem1):
    wid = lax.axis_index("s") * NC + lax.axis_index("c")
    sems = (sem0, sem1)
    base = wid * EL
    negs = (ng0, ng1, ng2, ng3, ng4, ng5, ng6, ng7)

    def idx_load(ref):
        pltpu.sync_copy(ref.at[:, pl.ds(wid * CT_PER_W, CT_PER_W)], idx_v)

    _emit_stripe_set(
        w_src,
        lambda: idx_load(srcT),
        lambda: pltpu.sync_copy(stage_v, su_out.at[pl.ds(base, EL)]),
        idx_v, rows_v, stage_v, sems)

    def tgt_set(s, _):
        def load():
            @pl.when(s == 0)
            def _():
                idx_load(posT)
            for n in range(NUM_NEG):
                @pl.when(s == n + 1)
                def _():
                    idx_load(negs[n])

        def store():
            @pl.when(s == 0)
            def _():
                pltpu.sync_copy(stage_v, tp_out.at[pl.ds(base, EL)])
            for n in range(NUM_NEG):
                @pl.when(s == n + 1)
                def _():
                    pltpu.sync_copy(stage_v,
                                    tn_out.at[n].at[pl.ds(base, EL)])

        _emit_stripe_set(w_tgt, load, store, idx_v, rows_v, stage_v, sems)
        return 0

    lax.fori_loop(0, 1 + NUM_NEG, tgt_set, 0)


def _make_sc_pool():
    mesh = plsc.VectorSubcoreMesh(core_axis_name="c", subcore_axis_name="s",
                                  num_cores=NC, num_subcores=NS)
    return pl.kernel(
        _sc_body,
        out_type=[
            jax.ShapeDtypeStruct((B, EMB_DIM), jnp.float32),
            jax.ShapeDtypeStruct((B, EMB_DIM), jnp.float32),
            jax.ShapeDtypeStruct((NUM_NEG, B, EMB_DIM), jnp.float32),
        ],
        mesh=mesh,
        scratch_types=[
            pltpu.VMEM((2, CT_PER_W, 8, 2, CH), jnp.int32),
            pltpu.VMEM((2, M, CH, EMB_DIM), jnp.float32),
            pltpu.VMEM((EL, EMB_DIM), jnp.float32),
            pltpu.SemaphoreType.DMA,
            pltpu.SemaphoreType.DMA,
        ],
        compiler_params=pltpu.CompilerParams(use_tc_tiling_on_sc=False),
    )


def _softplus(x):
    return jnp.maximum(x, 0.0) + jnp.log1p(jnp.exp(-jnp.abs(x)))


def _loss_body(su_ref, tp_ref, tn_ref, out_ref):
    su = su_ref[...]
    tp = tp_ref[...]
    acc = _softplus(-jnp.sum(su * tp, axis=1))
    for n in range(NUM_NEG):
        acc = acc + _softplus(jnp.sum(su * tn_ref[n], axis=1))
    tot = jnp.sum(acc).reshape(1, 1)

    @pl.when(pl.program_id(0) == 0)
    def _():
        out_ref[...] = jnp.zeros((1, 1), jnp.float32)

    out_ref[...] += tot


_TC_BLOCK = 512


def _make_tc_loss():
    grid = (B // _TC_BLOCK,)
    return pl.pallas_call(
        _loss_body,
        grid=grid,
        in_specs=[
            pl.BlockSpec((_TC_BLOCK, EMB_DIM), lambda i: (i, 0)),
            pl.BlockSpec((_TC_BLOCK, EMB_DIM), lambda i: (i, 0)),
            pl.BlockSpec((NUM_NEG, _TC_BLOCK, EMB_DIM), lambda i: (0, i, 0)),
        ],
        out_specs=pl.BlockSpec((1, 1), lambda i: (0, 0)),
        out_shape=jax.ShapeDtypeStruct((1, 1), jnp.float32),
    )


@jax.jit
def kernel(src_hashes, pos_dst_hashes, neg_dst_hashes, W_src, W_tgt):
    def native5(x2d):
        return (x2d.astype(jnp.int32).T.reshape(2, 8, BT, 128)
                .transpose(0, 2, 1, 3).reshape(2, BT, 8, 2, CH))

    srcT = native5(src_hashes)
    posT = native5(pos_dst_hashes)
    negTs = [native5(neg_dst_hashes[:, n, :]) for n in range(NUM_NEG)]

    su, tp, tn = _make_sc_pool()(srcT, posT, *negTs, W_src, W_tgt)
    tot = _make_tc_loss()(su, tp, tn)
    return tot[0, 0] / B
